# baseline (speedup 1.0000x reference)
import jax
import jax.numpy as jnp
from jax import lax
from jax.experimental import pallas as pl
from jax.experimental.pallas import tpu as pltpu
from jax.experimental.pallas import tpu_sc as plsc

ROWS = 100000
COLS = 64
G = 8
NGRP = ROWS // G
NIDX = 16384
NC = 2
NS = 16
NW = NC * NS
GPW = NGRP // NW
EXTRA = NGRP - NW * GPW
WIN = 1024
NWIN = NIDX // WIN
CAP = WIN + 16


def _body(x_hbm, idx_hbm, srcf_hbm, alpha_hbm, out_hbm,
          idxwin0, selrow, selpos, slots, rowbuf, scr, alphabuf,
          sem_i, sem_a, sem_b, sem_s):
    c = lax.axis_index("c")
    t = lax.axis_index("s")
    w = c * NS + t
    lanes = lax.iota(jnp.int32, 16)

    pltpu.sync_copy(alpha_hbm, alphabuf)
    alpha_v = alphabuf[...]

    row_lo = w * GPW * G
    row_hi = row_lo + GPW * G
    eg = NGRP - EXTRA + w

    h_copy = pltpu.async_copy(x_hbm.at[pl.ds(w * GPW, GPW), :, :],
                              out_hbm.at[pl.ds(w * GPW, GPW), :, :], sem_s)

    @pl.when(w < EXTRA)
    def _():
        pltpu.sync_copy(x_hbm.at[pl.ds(eg, 1), :, :],
                        out_hbm.at[pl.ds(eg, 1), :, :])

    h_copy.wait()

    def window_step(win, carry0):
        idxwin = idxwin0
        pltpu.sync_copy(idx_hbm.at[pl.ds(win * WIN, WIN)], idxwin)

        def scan_step(j, cnt):
            v = idxwin[pl.ds(j * 16, 16)]
            m = (v >= row_lo) & (v < row_hi)
            if EXTRA:
                vg = lax.shift_right_logical(v, 3)
                m = m | ((w < EXTRA) & (vg == eg))
            plsc.store_compressed(selrow.at[pl.ds(cnt, 16)], v, mask=m)
            plsc.store_compressed(selpos.at[pl.ds(cnt, 16)],
                                  win * WIN + j * 16 + lanes, mask=m)
            return cnt + jnp.sum(m.astype(jnp.int32))
        cnt = lax.fori_loop(0, WIN // 16, scan_step, jnp.int32(0))

        selrow[pl.ds(cnt, 16)] = jnp.full((16,), 1, jnp.int32) * row_lo
        selpos[pl.ds(cnt, 16)] = lanes

        nbatches = (cnt + 15) // 16

        def batch_step(b, carry):
            rowv = selrow[pl.ds(b * 16, 16)]
            posv = selpos[pl.ds(b * 16, 16)]
            gv = lax.shift_right_logical(rowv, 3)
            rv = rowv & 7
            valid = cnt - b * 16

            hv = gv
            plsc.store_scatter(scr, [hv], lanes)
            back = plsc.load_gather(scr, [hv])
            nconf = jnp.sum((back != lanes).astype(jnp.int32))

            ha, hb = [], []
            for l in range(16):
                ha.append(pltpu.async_copy(out_hbm.at[pl.ds(gv[l], 1), :, :],
                                           slots.at[pl.ds(l, 1), :, :], sem_a))
                hb.append(pltpu.async_copy(
                    srcf_hbm.at[pl.ds(posv[l] * COLS, COLS)],
                    rowbuf.at[pl.ds(l * COLS, COLS)], sem_b))
            for h in ha:
                h.wait()
            for h in hb:
                h.wait()

            @pl.when(nconf == 0)
            def _():
                for l in range(16):
                    rs = rv[l]

                    @pl.when(l < valid)
                    def _():
                        for q in range(COLS // 16):
                            slots[l, rs, pl.ds(q * 16, 16)] = (
                                slots[l, rs, pl.ds(q * 16, 16)]
                                + rowbuf[pl.ds(l * COLS + q * 16, 16)] * alpha_v)
                hs = []
                for l in range(16):
                    hs.append(pltpu.async_copy(slots.at[pl.ds(l, 1), :, :],
                                               out_hbm.at[pl.ds(gv[l], 1), :, :],
                                               sem_s))
                for h in hs:
                    h.wait()

            @pl.when(nconf != 0)
            def _():
                for l in range(16):
                    rs = rv[l]

                    @pl.when(l < valid)
                    def _():
                        pltpu.sync_copy(out_hbm.at[pl.ds(gv[l], 1), :, :],
                                        slots.at[pl.ds(0, 1), :, :])
                        for q in range(COLS // 16):
                            slots[0, rs, pl.ds(q * 16, 16)] = (
                                slots[0, rs, pl.ds(q * 16, 16)]
                                + rowbuf[pl.ds(l * COLS + q * 16, 16)] * alpha_v)
                        pltpu.sync_copy(slots.at[pl.ds(0, 1), :, :],
                                        out_hbm.at[pl.ds(gv[l], 1), :, :])
            return carry
        lax.fori_loop(0, nbatches, batch_step, jnp.int32(0))
        return carry0

    lax.fori_loop(0, NWIN, window_step, jnp.int32(0))


def kernel(x, dim, index, source, alpha):
    idx32 = (index + dim).astype(jnp.int32)
    x3 = x.reshape(NGRP, G, COLS)
    srcf = source.reshape(-1)
    alpha_arr = jnp.full((16,), alpha, dtype=jnp.float32)

    mesh = plsc.VectorSubcoreMesh(core_axis_name="c", subcore_axis_name="s")
    f = pl.kernel(
        _body,
        mesh=mesh,
        compiler_params=pltpu.CompilerParams(needs_layout_passes=False,
                                             use_tc_tiling_on_sc=True),
        out_type=jax.ShapeDtypeStruct((NGRP, G, COLS), jnp.float32),
        scratch_types=[
            pltpu.VMEM((WIN,), jnp.int32),
            pltpu.VMEM((CAP,), jnp.int32),
            pltpu.VMEM((CAP,), jnp.int32),
            pltpu.VMEM((16, G, COLS), jnp.float32),
            pltpu.VMEM((16 * COLS,), jnp.float32),
            pltpu.VMEM((NGRP,), jnp.int32),
            pltpu.VMEM((16,), jnp.float32),
            pltpu.SemaphoreType.DMA,
            pltpu.SemaphoreType.DMA,
            pltpu.SemaphoreType.DMA,
            pltpu.SemaphoreType.DMA,
        ],
    )
    out3 = f(x3, idx32, srcf, alpha_arr)
    return out3.reshape(ROWS, COLS)


# speedup vs baseline: 7.7824x; 7.7824x over previous
import jax
import jax.numpy as jnp
from jax import lax
from jax.experimental import pallas as pl
from jax.experimental.pallas import tpu as pltpu
from jax.experimental.pallas import tpu_sc as plsc

ROWS = 100000
COLS = 64
PROWS = ROWS // 2
PCOLS = 2 * COLS
NIDX = 16384
NSRC_P = NIDX // 2
NC = 2
NS = 16
PHASES = 5
PIECE = PROWS // (NC * PHASES)
TROWS = 312
REM = PIECE - NS * TROWS
IDX_PER_TILE = NIDX // NS
CH = 128
PAD = CH + 16
TRASH = 8


def _body(x_hbm, idx_hbm, src_hbm, alpha_hbm, out_hbm,
          idxbuf, selrow, selpos, poschunk, locchunk, gbuf, stagebuf,
          alphabuf, piece, sem_g):
    c = lax.axis_index("c")
    t = lax.axis_index("s")
    lanes = lax.iota(jnp.int32, 16)
    zeros16 = jnp.zeros((16,), jnp.float32)

    pltpu.sync_copy(idx_hbm.at[pl.ds(t * IDX_PER_TILE, IDX_PER_TILE)], idxbuf)
    pltpu.sync_copy(alpha_hbm, alphabuf)
    alpha_v = alphabuf[...]

    for p in range(PHASES):
        base_p = (c * PHASES + p) * PIECE
        base_r = base_p * 2
        if p > 0:
            plsc.subcore_barrier()

        pltpu.sync_copy(x_hbm.at[pl.ds(base_p + t * TROWS, TROWS), :],
                        piece.at[pl.ds(t * TROWS, TROWS), :])

        @pl.when(t == 0)
        def _():
            pltpu.sync_copy(x_hbm.at[pl.ds(base_p + NS * TROWS, REM), :],
                            piece.at[pl.ds(NS * TROWS, REM), :])

        def scan_step(j, cnt):
            v = idxbuf[pl.ds(j * 16, 16)]
            m = (v >= base_r) & (v < base_r + 2 * PIECE)
            plsc.store_compressed(selrow.at[pl.ds(cnt, 16)], v - base_r, mask=m)
            plsc.store_compressed(selpos.at[pl.ds(cnt, 16)],
                                  t * IDX_PER_TILE + j * 16 + lanes, mask=m)
            return cnt + jnp.sum(m.astype(jnp.int32))
        cnt = lax.fori_loop(0, IDX_PER_TILE // 16, scan_step, jnp.int32(0))

        for k in range(CH // 16 + 1):
            selrow[pl.ds(cnt + k * 16, 16)] = 2 * PIECE + (lanes & 7) * 2
            selpos[pl.ds(cnt + k * 16, 16)] = lanes

        plsc.subcore_barrier()

        nchunks = (cnt + (CH - 1)) // CH

        def chunk_step(ci, carry):
            for k in range(CH // 16):
                rv = selrow[pl.ds(ci * CH + k * 16, 16)]
                pv = selpos[pl.ds(ci * CH + k * 16, 16)]
                locchunk[pl.ds(k * 16, 16)] = lax.shift_right_logical(rv, 1)
                poschunk[pl.ds(k * 16, 16)] = lax.shift_right_logical(pv, 1)
            pltpu.async_copy(src_hbm.at[poschunk], gbuf, sem_g).wait()

            def build_entry(e, carry2):
                rv1 = selrow[pl.ds(ci * CH + e, 16)][0]
                pv1 = selpos[pl.ds(ci * CH + e, 16)][0]
                dh = (rv1 & 1) * COLS
                sh = (pv1 & 1) * COLS
                for q in range(COLS // 16):
                    stagebuf[e, pl.ds(dh + q * 16, 16)] = (
                        gbuf[e, pl.ds(sh + q * 16, 16)] * alpha_v)
                    stagebuf[e, pl.ds((COLS - dh) + q * 16, 16)] = zeros16
                return carry2
            lax.fori_loop(0, CH, build_entry, jnp.int32(0))

            pltpu.sync_copy(stagebuf, piece.at[locchunk], add=True)
            return carry
        lax.fori_loop(0, nchunks, chunk_step, jnp.int32(0))

        plsc.subcore_barrier()

        pltpu.sync_copy(piece.at[pl.ds(t * TROWS, TROWS), :],
                        out_hbm.at[pl.ds(base_p + t * TROWS, TROWS), :])

        @pl.when(t == 0)
        def _():
            pltpu.sync_copy(piece.at[pl.ds(NS * TROWS, REM), :],
                            out_hbm.at[pl.ds(base_p + NS * TROWS, REM), :])


def kernel(x, dim, index, source, alpha):
    idx32 = (index + dim).astype(jnp.int32)
    x2 = x.reshape(PROWS, PCOLS)
    src2 = source.reshape(NSRC_P, PCOLS)
    alpha_arr = jnp.full((16,), alpha, dtype=jnp.float32)

    mesh = plsc.VectorSubcoreMesh(core_axis_name="c", subcore_axis_name="s")
    f = pl.kernel(
        _body,
        mesh=mesh,
        compiler_params=pltpu.CompilerParams(needs_layout_passes=False,
                                             use_tc_tiling_on_sc=True),
        out_type=jax.ShapeDtypeStruct((PROWS, PCOLS), jnp.float32),
        scratch_types=[
            pltpu.VMEM((IDX_PER_TILE,), jnp.int32),
            pltpu.VMEM((IDX_PER_TILE + PAD,), jnp.int32),
            pltpu.VMEM((IDX_PER_TILE + PAD,), jnp.int32),
            pltpu.VMEM((CH,), jnp.int32),
            pltpu.VMEM((CH,), jnp.int32),
            pltpu.VMEM((CH, PCOLS), jnp.float32),
            pltpu.VMEM((CH, PCOLS), jnp.float32),
            pltpu.VMEM((16,), jnp.float32),
            pltpu.VMEM_SHARED((PIECE + TRASH, PCOLS), jnp.float32),
            pltpu.SemaphoreType.DMA,
        ],
    )
    out2 = f(x2, idx32, src2, alpha_arr)
    return out2.reshape(ROWS, COLS)
